# baseline (speedup 1.0000x reference)
import functools

import jax
import jax.numpy as jnp
from jax import lax
from jax.experimental import pallas as pl
from jax.experimental.pallas import tpu as pltpu
from jax.experimental.pallas import tpu_sc as plsc

N_NODES = 100000
D_FEAT = 512
NUM_SEGMENTS = 1024

NC = 2
NS = 16
NW = NC * NS
SEGS_PER_W = NUM_SEGMENTS // NW
R = 64
NBUF = 3
LAST_BASE = N_NODES - R
NSCAN = 8
SLICE = 12512
SLICE_LAST = N_NODES - (NSCAN - 1) * SLICE
LANES = 16
NWIN = SLICE // LANES
NPROBE = 10
SENTINEL = 2 * NUM_SEGMENTS
CPR = D_FEAT // LANES
NB = SEGS_PER_W + 1

_mesh = plsc.VectorSubcoreMesh(core_axis_name="c", subcore_axis_name="s")


@functools.partial(
    pl.kernel,
    out_type=jax.ShapeDtypeStruct((NUM_SEGMENTS, D_FEAT), jnp.float32),
    mesh=_mesh,
    compiler_params=pltpu.CompilerParams(needs_layout_passes=False),
    scratch_types=[
        pltpu.VMEM((R, D_FEAT), jnp.float32),
        pltpu.VMEM((R, D_FEAT), jnp.float32),
        pltpu.VMEM((R, D_FEAT), jnp.float32),
        pltpu.VMEM((SEGS_PER_W, D_FEAT), jnp.float32),
        pltpu.VMEM((SLICE,), jnp.int32),
        pltpu.VMEM((NB * LANES,), jnp.int32),
        pltpu.SemaphoreType.DMA,
        pltpu.SemaphoreType.DMA,
        pltpu.SemaphoreType.DMA,
    ],
)
def _sc_segment_sum(x_hbm, idx_hbm, out_hbm, x_v0, x_v1, x_v2, acc, sl_v,
                    b_v, sem0, sem1, sem2):
    c = lax.axis_index("c")
    s = lax.axis_index("s")
    w = c * NS + s

    t_lo = w * SEGS_PER_W
    lane = lax.iota(jnp.int32, LANES)
    zi = jnp.zeros((LANES,), jnp.int32)
    sent = jnp.full((LANES,), SENTINEL, jnp.int32)

    for m in range(NB):
        b_v[pl.ds(m * LANES, LANES)] = zi

    def scan_chunk(ci, carry):
        cbase = pl.multiple_of(ci * SLICE, 8)

        @pl.when(ci < NSCAN - 1)
        def _():
            pltpu.sync_copy(idx_hbm.at[pl.ds(cbase, SLICE)], sl_v)

        @pl.when(ci == NSCAN - 1)
        def _():
            pltpu.sync_copy(
                idx_hbm.at[pl.ds((NSCAN - 1) * SLICE, SLICE_LAST)],
                sl_v.at[pl.ds(0, SLICE_LAST)])
            for k in range(SLICE_LAST // LANES, SLICE // LANES):
                sl_v[pl.ds(k * LANES, LANES)] = sent

        n_valid = jnp.where(ci == NSCAN - 1, SLICE_LAST, SLICE)
        first = sl_v[pl.ds(0, LANES)][0]
        last_full = sl_v[pl.ds(SLICE - LANES, LANES)][LANES - 1]
        last_part = sl_v[pl.ds(SLICE_LAST - LANES, LANES)][LANES - 1]
        last = jnp.where(ci == NSCAN - 1, last_part, last_full)

        def bnd(m, carry2):
            t = t_lo + m
            simple = jnp.where(last < t, n_valid, 0)
            b_v[pl.ds(m * LANES, LANES)] = (
                b_v[pl.ds(m * LANES, LANES)] + jnp.full((LANES,), simple,
                                                        jnp.int32))

            @pl.when((first < t) & (t <= last))
            def _():
                one = jnp.ones((LANES,), jnp.int32)

                def cwin(k, cc):
                    v = sl_v[pl.ds(k * LANES, LANES)]
                    return cc + jnp.where(v < t, one, zi)

                cvec = lax.fori_loop(0, NWIN, cwin, zi)
                cnt = jnp.int32(0)
                for i in range(LANES):
                    cnt = cnt + cvec[i]
                b_v[pl.ds(m * LANES, LANES)] = (
                    b_v[pl.ds(m * LANES, LANES)] + jnp.full((LANES,), cnt,
                                                            jnp.int32))

            return carry2

        lax.fori_loop(0, NB, bnd, 0)
        return carry

    lax.fori_loop(0, NSCAN, scan_chunk, 0)

    lo = b_v[pl.ds(0, LANES)][0]
    hi = b_v[pl.ds(SEGS_PER_W * LANES, LANES)][0]

    zf = jnp.zeros((LANES,), jnp.float32)

    def zrow(i, carry):
        acc[i // CPR, pl.ds((i % CPR) * LANES, LANES)] = zf
        return carry

    lax.fori_loop(0, SEGS_PER_W * CPR, zrow, 0)

    def chunk_base(nxt):
        return pl.multiple_of(jnp.minimum(nxt & ~7, LAST_BASE), 8)

    del x_v1, x_v2, sem0, sem1, sem2

    def process(nxt, base, ce, buf):
        def seg(m, carry2):
            bm = b_v[pl.ds(m * LANES, LANES)][0]
            bm1 = b_v[pl.ds((m + 1) * LANES, LANES)][0]
            p = jnp.maximum(bm, nxt) - base
            q = jnp.minimum(bm1, ce) - base

            @pl.when(q > p)
            def _():
                def row(r, regs):
                    return tuple(
                        regs[g] + buf[r, pl.ds(g * LANES, LANES)]
                        for g in range(CPR))

                regs0 = (zf,) * CPR
                sums = lax.fori_loop(p, q, row, regs0)
                for g in range(CPR):
                    cur = acc[m, pl.ds(g * LANES, LANES)]
                    acc[m, pl.ds(g * LANES, LANES)] = cur + sums[g]

            return carry2

        lax.fori_loop(0, SEGS_PER_W, seg, 0)

    def body(j, carry):
        del j
        nxt = carry
        active = nxt < hi
        base = chunk_base(nxt)
        ce = jnp.minimum(base + R, hi)

        @pl.when(active)
        def _():
            pltpu.sync_copy(x_hbm.at[pl.ds(base, R)], x_v0)
            process(nxt, base, ce, x_v0)

        return jnp.where(active, ce, nxt)

    n_it = (hi - lo + R - 8) // (R - 7) + 1
    lax.fori_loop(0, n_it, body, lo)

    pltpu.sync_copy(acc, out_hbm.at[pl.ds(t_lo, SEGS_PER_W)])


@jax.jit
def kernel(x, molecule_idx):
    return _sc_segment_sum(x, molecule_idx.astype(jnp.int32))
